# baseline (speedup 1.0000x reference)
import functools

import jax
import jax.numpy as jnp
import numpy as np
from jax import lax
from jax.experimental import pallas as pl
from jax.experimental.pallas import tpu as pltpu
from jax.experimental.pallas import tpu_sc as plsc

B = 32
N = 1_000_000
K = 32

BLK = 1024
SPAN = 8192
NSPAN = 123
RSTRIDE = NSPAN * SPAN
NBLK = NSPAN * 8
MBLK = 992
TAIL_W = N - 122 * SPAN

POOL = 256
LIMIT = POOL - 16
PV = POOL // 16
MV = MBLK // 16

NEG = np.float32(-np.inf)
IMAX = np.int32(2**31 - 1)


def _prep_body(x_ref, f_ref, m_ref):
    c = pl.program_id(1)
    x = x_ref[...]
    col = lax.broadcasted_iota(jnp.int32, (8, SPAN), 1)
    x = jnp.where((c < NSPAN - 1) | (col < TAIL_W), x, NEG)
    f_ref[...] = x.reshape(8, 1, 64, 128)
    m_ref[...] = jnp.max(x.reshape(8, 8, BLK), axis=-1)[None]


_prep_call = pl.pallas_call(
    _prep_body,
    grid=(B // 8, NSPAN),
    in_specs=[pl.BlockSpec((8, SPAN), lambda rg, c: (rg, c))],
    out_specs=[
        pl.BlockSpec((8, 1, 64, 128), lambda rg, c: (rg, c, 0, 0)),
        pl.BlockSpec((1, 8, 8), lambda rg, c: (c, rg, 0)),
    ],
    out_shape=[
        jax.ShapeDtypeStruct((B, NSPAN, 64, 128), jnp.float32),
        jax.ShapeDtypeStruct((NSPAN, B, 8), jnp.float32),
    ],
)


def _select_body(flat_hbm, mflat_hbm, out_idx_hbm, out_prob_hbm,
                 mrow_ref, gath_ref, pool_val, pool_idx, wv_ref, wi_ref,
                 prob_buf, t_ref, cnt_ref, bid_ref, sem):
    nc = 2
    wid = lax.axis_index("s") * nc + lax.axis_index("c")
    iota = lax.iota(jnp.int32, 16)
    lane0 = iota == 0

    def extract32():
        def round_body(k, _):
            def pa(i, mm):
                return jnp.maximum(mm, jnp.max(pool_val[pl.ds(i * 16, 16)]))
            m = lax.fori_loop(0, PV, pa, NEG)

            def pb(i, jm):
                pv = pool_val[pl.ds(i * 16, 16)]
                pi = pool_idx[pl.ds(i * 16, 16)]
                cand = jnp.where(pv == m, pi, IMAX)
                return jnp.minimum(jm, jnp.min(cand))
            jmin = lax.fori_loop(0, PV, pb, IMAX)

            def pc(i, c):
                pv = pool_val[pl.ds(i * 16, 16)]
                pi = pool_idx[pl.ds(i * 16, 16)]
                pool_val[pl.ds(i * 16, 16)] = jnp.where(pi == jmin, NEG, pv)
                return c
            lax.fori_loop(0, PV, pc, 0)
            kv = jnp.full((16,), k, jnp.int32)
            plsc.store_scatter(wv_ref, [kv], jnp.full((16,), m, jnp.float32),
                               mask=lane0)
            plsc.store_scatter(wi_ref, [kv], jnp.full((16,), jmin, jnp.int32),
                               mask=lane0)
            return _
        lax.fori_loop(0, K, round_body, 0)

    def compact():
        extract32()
        for h in range(2):
            pool_val[pl.ds(h * 16, 16)] = wv_ref[pl.ds(h * 16, 16)]
            pool_idx[pl.ds(h * 16, 16)] = wi_ref[pl.ds(h * 16, 16)]

        def clear(i, c):
            pool_val[pl.ds(32 + i * 16, 16)] = jnp.full((16,), NEG, jnp.float32)
            return c
        lax.fori_loop(0, PV - 2, clear, 0)
        cnt_ref[0] = jnp.int32(K)
        t_ref[0] = wv_ref[pl.ds(K - 16, 16)][15]

    def process_vec(off, idx_base):
        v = gath_ref[pl.ds(off, 16)]
        m = v > t_ref[0]
        c = jnp.sum(m.astype(jnp.int32))

        @pl.when(c > 0)
        def _():
            cnt = cnt_ref[0]
            pos = cnt - 1 + plsc.cumsum(m.astype(jnp.int32))
            plsc.store_scatter(pool_val, [pos], v, mask=m)
            iv = idx_base + iota
            plsc.store_scatter(pool_idx, [pos], iv, mask=m)
            cnt_ref[0] = cnt + c

            @pl.when(cnt + c >= LIMIT)
            def _():
                compact()

    def scan_group(off, idx_base):
        gm = gath_ref[pl.ds(off, 16)]
        for j in range(1, 8):
            gm = jnp.maximum(gm, gath_ref[pl.ds(off + j * 16, 16)])

        @pl.when(jnp.max(gm) > t_ref[0])
        def _():
            for j in range(8):
                process_vec(off + j * 16, idx_base + j * 16)

    @pl.when(wid < B)
    def _():
        row_off = wid * RSTRIDE
        pltpu.sync_copy(mflat_hbm.at[pl.ds(wid * MBLK, MBLK)], mrow_ref)

        def bid_round(k, _):
            def pa(i, mm):
                return jnp.maximum(mm, jnp.max(mrow_ref[pl.ds(i * 16, 16)]))
            m = lax.fori_loop(0, MV, pa, NEG)

            def pb(i, jm):
                rv = mrow_ref[pl.ds(i * 16, 16)]
                cand = jnp.where(rv == m, i * 16 + iota, IMAX)
                return jnp.minimum(jm, jnp.min(cand))
            jmin = lax.fori_loop(0, MV, pb, IMAX)
            plsc.store_scatter(mrow_ref, [jnp.full((16,), jmin, jnp.int32)],
                               jnp.full((16,), NEG, jnp.float32), mask=lane0)
            bid_ref[k] = jmin
            return _
        lax.fori_loop(0, K, bid_round, 0)

        cps = []
        for k in range(K):
            cps.append(pltpu.make_async_copy(
                flat_hbm.at[pl.ds(row_off + bid_ref[k] * BLK, BLK)],
                gath_ref.at[pl.ds(k * BLK, BLK)], sem))
        waves = [cps[i:i + 8] for i in range(0, len(cps), 8)]
        for cp in waves[0]:
            cp.start()
        for w in range(1, len(waves)):
            for cp in waves[w]:
                cp.start()
            for cp in waves[w - 1]:
                cp.wait()
        for cp in waves[-1]:
            cp.wait()

        def init(i, c):
            pool_val[pl.ds(i * 16, 16)] = jnp.full((16,), NEG, jnp.float32)
            pool_idx[pl.ds(i * 16, 16)] = jnp.zeros((16,), jnp.int32)
            return c
        lax.fori_loop(0, PV, init, 0)
        cnt_ref[0] = jnp.int32(0)
        t_ref[0] = NEG

        def blk_body(k, carry):
            base = bid_ref[k] * BLK

            def grp(g, gc):
                scan_group(k * BLK + g * 128, base + g * 128)
                return gc
            lax.fori_loop(0, 8, grp, 0)
            return carry
        lax.fori_loop(0, K, blk_body, 0)

        extract32()
        v0 = wv_ref[pl.ds(0, 16)]
        v1 = wv_ref[pl.ds(16, 16)]
        mtop = v0[0]
        e0 = jnp.exp(v0 - mtop)
        e1 = jnp.exp(v1 - mtop)
        s = jnp.sum(e0) + jnp.sum(e1)
        prob_buf[pl.ds(0, 16)] = e0 / s
        prob_buf[pl.ds(16, 16)] = e1 / s
        pltpu.sync_copy(wi_ref, out_idx_hbm.at[pl.ds(wid * K, K)])
        pltpu.sync_copy(prob_buf, out_prob_hbm.at[pl.ds(wid * K, K)])


_mesh = plsc.VectorSubcoreMesh(core_axis_name="c", subcore_axis_name="s")

_select_call = functools.partial(
    pl.kernel,
    mesh=_mesh,
    compiler_params=pltpu.CompilerParams(needs_layout_passes=False),
    out_type=[
        jax.ShapeDtypeStruct((B * K,), jnp.int32),
        jax.ShapeDtypeStruct((B * K,), jnp.float32),
    ],
    scratch_types=[
        pltpu.VMEM((MBLK,), jnp.float32),
        pltpu.VMEM((K * BLK,), jnp.float32),
        pltpu.VMEM((POOL,), jnp.float32),
        pltpu.VMEM((POOL,), jnp.int32),
        pltpu.VMEM((K,), jnp.float32),
        pltpu.VMEM((K,), jnp.int32),
        pltpu.VMEM((K,), jnp.float32),
        pltpu.SMEM((1,), jnp.float32),
        pltpu.SMEM((1,), jnp.int32),
        pltpu.SMEM((K,), jnp.int32),
        pltpu.SemaphoreType.DMA,
    ],
)(_select_body)


def kernel(logits):
    flat4, m = _prep_call(logits)
    flat = flat4.reshape(-1)
    mrow = m.transpose(1, 0, 2).reshape(B, NBLK)
    mrow = jnp.pad(mrow, ((0, 0), (0, MBLK - NBLK)),
                   constant_values=np.float32(-np.inf))
    idx_flat, prob_flat = _select_call(flat, mrow.reshape(-1))
    return idx_flat.reshape(B, K), prob_flat.reshape(B, K)


# speedup vs baseline: 1.4062x; 1.4062x over previous
import functools

import jax
import jax.numpy as jnp
import numpy as np
from jax import lax
from jax.experimental import pallas as pl
from jax.experimental.pallas import tpu as pltpu
from jax.experimental.pallas import tpu_sc as plsc

B = 32
N = 1_000_000
K = 32

BLK = 1024
SPAN = 16384
NSPAN = 62
RSTRIDE = NSPAN * SPAN
NBLK = NSPAN * 16
MBLK = 992
TAIL_W = N - 61 * SPAN

POOL = 256
LIMIT = POOL - 16
PV = POOL // 16
MV = MBLK // 16

NEG = np.float32(-np.inf)
IMAX = np.int32(2**31 - 1)


def _prep_body(x_ref, f_ref, m_ref):
    c = pl.program_id(1)
    x = x_ref[...]
    col = lax.broadcasted_iota(jnp.int32, (8, SPAN), 1)
    x = jnp.where((c < NSPAN - 1) | (col < TAIL_W), x, NEG)
    f_ref[...] = x.reshape(8, 1, 128, 128)
    m_ref[...] = jnp.max(x.reshape(8, 16, BLK), axis=-1)[None]


_prep_call = pl.pallas_call(
    _prep_body,
    grid=(B // 8, NSPAN),
    in_specs=[pl.BlockSpec((8, SPAN), lambda rg, c: (rg, c))],
    out_specs=[
        pl.BlockSpec((8, 1, 128, 128), lambda rg, c: (rg, c, 0, 0)),
        pl.BlockSpec((1, 8, 16), lambda rg, c: (c, rg, 0)),
    ],
    out_shape=[
        jax.ShapeDtypeStruct((B, NSPAN, 128, 128), jnp.float32),
        jax.ShapeDtypeStruct((NSPAN, B, 16), jnp.float32),
    ],
)


def _select_body(flat_hbm, mflat_hbm, out_idx_hbm, out_prob_hbm,
                 mrow_ref, gath_ref, pool_val, pool_idx, wv_ref, wi_ref,
                 prob_buf, t_ref, cnt_ref, bid_ref, sem):
    nc = 2
    wid = lax.axis_index("s") * nc + lax.axis_index("c")
    iota = lax.iota(jnp.int32, 16)
    lane0 = iota == 0

    def extract32():
        def round_body(k, _):
            def pa(i, mm):
                return jnp.maximum(mm, jnp.max(pool_val[pl.ds(i * 16, 16)]))
            m = lax.fori_loop(0, PV, pa, NEG)

            def pb(i, jm):
                pv = pool_val[pl.ds(i * 16, 16)]
                pi = pool_idx[pl.ds(i * 16, 16)]
                cand = jnp.where(pv == m, pi, IMAX)
                return jnp.minimum(jm, jnp.min(cand))
            jmin = lax.fori_loop(0, PV, pb, IMAX)

            def pc(i, c):
                pv = pool_val[pl.ds(i * 16, 16)]
                pi = pool_idx[pl.ds(i * 16, 16)]
                pool_val[pl.ds(i * 16, 16)] = jnp.where(pi == jmin, NEG, pv)
                return c
            lax.fori_loop(0, PV, pc, 0)
            kv = jnp.full((16,), k, jnp.int32)
            plsc.store_scatter(wv_ref, [kv], jnp.full((16,), m, jnp.float32),
                               mask=lane0)
            plsc.store_scatter(wi_ref, [kv], jnp.full((16,), jmin, jnp.int32),
                               mask=lane0)
            return _
        lax.fori_loop(0, K, round_body, 0)

    def compact():
        extract32()
        for h in range(2):
            pool_val[pl.ds(h * 16, 16)] = wv_ref[pl.ds(h * 16, 16)]
            pool_idx[pl.ds(h * 16, 16)] = wi_ref[pl.ds(h * 16, 16)]

        def clear(i, c):
            pool_val[pl.ds(32 + i * 16, 16)] = jnp.full((16,), NEG, jnp.float32)
            return c
        lax.fori_loop(0, PV - 2, clear, 0)
        cnt_ref[0] = jnp.int32(K)
        t_ref[0] = wv_ref[pl.ds(K - 16, 16)][15]

    def process_vec(off, idx_base):
        v = gath_ref[pl.ds(off, 16)]
        m = v > t_ref[0]
        c = jnp.sum(m.astype(jnp.int32))

        @pl.when(c > 0)
        def _():
            cnt = cnt_ref[0]
            pos = cnt - 1 + plsc.cumsum(m.astype(jnp.int32))
            plsc.store_scatter(pool_val, [pos], v, mask=m)
            iv = idx_base + iota
            plsc.store_scatter(pool_idx, [pos], iv, mask=m)
            cnt_ref[0] = cnt + c

            @pl.when(cnt + c >= LIMIT)
            def _():
                compact()

    def scan_group(off, idx_base):
        gm = gath_ref[pl.ds(off, 16)]
        for j in range(1, 8):
            gm = jnp.maximum(gm, gath_ref[pl.ds(off + j * 16, 16)])

        @pl.when(jnp.max(gm) > t_ref[0])
        def _():
            for j in range(8):
                process_vec(off + j * 16, idx_base + j * 16)

    @pl.when(wid < B)
    def _():
        row_off = wid * RSTRIDE
        pltpu.sync_copy(mflat_hbm.at[pl.ds(wid * MBLK, MBLK)], mrow_ref)

        def bid_round(k, _):
            def pa(i, mm):
                return jnp.maximum(mm, jnp.max(mrow_ref[pl.ds(i * 16, 16)]))
            m = lax.fori_loop(0, MV, pa, NEG)

            def pb(i, jm):
                rv = mrow_ref[pl.ds(i * 16, 16)]
                cand = jnp.where(rv == m, i * 16 + iota, IMAX)
                return jnp.minimum(jm, jnp.min(cand))
            jmin = lax.fori_loop(0, MV, pb, IMAX)
            plsc.store_scatter(mrow_ref, [jnp.full((16,), jmin, jnp.int32)],
                               jnp.full((16,), NEG, jnp.float32), mask=lane0)
            bid_ref[k] = jmin
            return _
        lax.fori_loop(0, K, bid_round, 0)

        cps = []
        for k in range(K):
            cps.append(pltpu.make_async_copy(
                flat_hbm.at[pl.ds(row_off + bid_ref[k] * BLK, BLK)],
                gath_ref.at[pl.ds(k * BLK, BLK)], sem))
        waves = [cps[i:i + 8] for i in range(0, len(cps), 8)]
        for cp in waves[0]:
            cp.start()
        for w in range(1, len(waves)):
            for cp in waves[w]:
                cp.start()
            for cp in waves[w - 1]:
                cp.wait()
        for cp in waves[-1]:
            cp.wait()

        def init(i, c):
            pool_val[pl.ds(i * 16, 16)] = jnp.full((16,), NEG, jnp.float32)
            pool_idx[pl.ds(i * 16, 16)] = jnp.zeros((16,), jnp.int32)
            return c
        lax.fori_loop(0, PV, init, 0)
        cnt_ref[0] = jnp.int32(0)
        t_ref[0] = NEG

        def blk_body(k, carry):
            base = bid_ref[k] * BLK

            def grp(g, gc):
                scan_group(k * BLK + g * 128, base + g * 128)
                return gc
            lax.fori_loop(0, 8, grp, 0)
            return carry
        lax.fori_loop(0, K, blk_body, 0)

        extract32()
        v0 = wv_ref[pl.ds(0, 16)]
        v1 = wv_ref[pl.ds(16, 16)]
        mtop = v0[0]
        e0 = jnp.exp(v0 - mtop)
        e1 = jnp.exp(v1 - mtop)
        s = jnp.sum(e0) + jnp.sum(e1)
        prob_buf[pl.ds(0, 16)] = e0 / s
        prob_buf[pl.ds(16, 16)] = e1 / s
        pltpu.sync_copy(wi_ref, out_idx_hbm.at[pl.ds(wid * K, K)])
        pltpu.sync_copy(prob_buf, out_prob_hbm.at[pl.ds(wid * K, K)])


_mesh = plsc.VectorSubcoreMesh(core_axis_name="c", subcore_axis_name="s")

_select_call = functools.partial(
    pl.kernel,
    mesh=_mesh,
    compiler_params=pltpu.CompilerParams(needs_layout_passes=False),
    out_type=[
        jax.ShapeDtypeStruct((B * K,), jnp.int32),
        jax.ShapeDtypeStruct((B * K,), jnp.float32),
    ],
    scratch_types=[
        pltpu.VMEM((MBLK,), jnp.float32),
        pltpu.VMEM((K * BLK,), jnp.float32),
        pltpu.VMEM((POOL,), jnp.float32),
        pltpu.VMEM((POOL,), jnp.int32),
        pltpu.VMEM((K,), jnp.float32),
        pltpu.VMEM((K,), jnp.int32),
        pltpu.VMEM((K,), jnp.float32),
        pltpu.SMEM((1,), jnp.float32),
        pltpu.SMEM((1,), jnp.int32),
        pltpu.SMEM((K,), jnp.int32),
        pltpu.SemaphoreType.DMA,
    ],
)(_select_body)


def kernel(logits):
    flat4, m = _prep_call(logits)
    flat = flat4.reshape(-1)
    mrow = m.transpose(1, 0, 2).reshape(B, NBLK)
    idx_flat, prob_flat = _select_call(flat, mrow.reshape(-1))
    return idx_flat.reshape(B, K), prob_flat.reshape(B, K)


# speedup vs baseline: 1.7811x; 1.2665x over previous
import functools

import jax
import jax.numpy as jnp
import numpy as np
from jax import lax
from jax.experimental import pallas as pl
from jax.experimental.pallas import tpu as pltpu
from jax.experimental.pallas import tpu_sc as plsc

B = 32
N = 1_000_000
K = 32

BLK = 1024
SPAN = 32768
NSPAN = 31
RSTRIDE = NSPAN * SPAN
NBLK = NSPAN * 32
MBLK = 992
TAIL_W = N - 30 * SPAN

POOL = 256
LIMIT = POOL - 16
PV = POOL // 16
MV = MBLK // 16

NEG = np.float32(-np.inf)
IMAX = np.int32(2**31 - 1)


def _prep_body(x_ref, f_ref, m_ref):
    c = pl.program_id(1)
    x = x_ref[...]
    col = lax.broadcasted_iota(jnp.int32, (8, SPAN), 1)
    x = jnp.where((c < NSPAN - 1) | (col < TAIL_W), x, NEG)
    f_ref[...] = x.reshape(8, 1, 256, 128)
    m_ref[...] = jnp.max(x.reshape(8, 32, BLK), axis=-1)[None]


_prep_call = pl.pallas_call(
    _prep_body,
    grid=(B // 8, NSPAN),
    in_specs=[pl.BlockSpec((8, SPAN), lambda rg, c: (rg, c))],
    out_specs=[
        pl.BlockSpec((8, 1, 256, 128), lambda rg, c: (rg, c, 0, 0)),
        pl.BlockSpec((1, 8, 32), lambda rg, c: (c, rg, 0)),
    ],
    out_shape=[
        jax.ShapeDtypeStruct((B, NSPAN, 256, 128), jnp.float32),
        jax.ShapeDtypeStruct((NSPAN, B, 32), jnp.float32),
    ],
)


def _select_body(flat_hbm, mflat_hbm, out_idx_hbm, out_prob_hbm,
                 mrow_ref, gath_ref, pool_val, pool_idx, wv_ref, wi_ref,
                 prob_buf, t_ref, cnt_ref, bid_ref, sem):
    nc = 2
    wid = lax.axis_index("s") * nc + lax.axis_index("c")
    iota = lax.iota(jnp.int32, 16)
    lane0 = iota == 0

    def extract32():
        def round_body(k, _):
            def pa(i, mm):
                return jnp.maximum(mm, jnp.max(pool_val[pl.ds(i * 16, 16)]))
            m = lax.fori_loop(0, PV, pa, NEG)

            def pb(i, jm):
                pv = pool_val[pl.ds(i * 16, 16)]
                pi = pool_idx[pl.ds(i * 16, 16)]
                cand = jnp.where(pv == m, pi, IMAX)
                return jnp.minimum(jm, jnp.min(cand))
            jmin = lax.fori_loop(0, PV, pb, IMAX)

            def pc(i, c):
                pv = pool_val[pl.ds(i * 16, 16)]
                pi = pool_idx[pl.ds(i * 16, 16)]
                pool_val[pl.ds(i * 16, 16)] = jnp.where(pi == jmin, NEG, pv)
                return c
            lax.fori_loop(0, PV, pc, 0)
            kv = jnp.full((16,), k, jnp.int32)
            plsc.store_scatter(wv_ref, [kv], jnp.full((16,), m, jnp.float32),
                               mask=lane0)
            plsc.store_scatter(wi_ref, [kv], jnp.full((16,), jmin, jnp.int32),
                               mask=lane0)
            return _
        lax.fori_loop(0, K, round_body, 0)

    def compact():
        extract32()
        for h in range(2):
            pool_val[pl.ds(h * 16, 16)] = wv_ref[pl.ds(h * 16, 16)]
            pool_idx[pl.ds(h * 16, 16)] = wi_ref[pl.ds(h * 16, 16)]

        def clear(i, c):
            pool_val[pl.ds(32 + i * 16, 16)] = jnp.full((16,), NEG, jnp.float32)
            return c
        lax.fori_loop(0, PV - 2, clear, 0)
        cnt_ref[0] = jnp.int32(K)
        t_ref[0] = wv_ref[pl.ds(K - 16, 16)][15]

    def process_vec(off, idx_base):
        v = gath_ref[pl.ds(off, 16)]
        m = v > t_ref[0]
        c = jnp.sum(m.astype(jnp.int32))

        @pl.when(c > 0)
        def _():
            cnt = cnt_ref[0]
            pos = cnt - 1 + plsc.cumsum(m.astype(jnp.int32))
            plsc.store_scatter(pool_val, [pos], v, mask=m)
            iv = idx_base + iota
            plsc.store_scatter(pool_idx, [pos], iv, mask=m)
            cnt_ref[0] = cnt + c

            @pl.when(cnt + c >= LIMIT)
            def _():
                compact()

    def scan_group(off, idx_base):
        gm = gath_ref[pl.ds(off, 16)]
        for j in range(1, 8):
            gm = jnp.maximum(gm, gath_ref[pl.ds(off + j * 16, 16)])

        @pl.when(jnp.max(gm) > t_ref[0])
        def _():
            for j in range(8):
                process_vec(off + j * 16, idx_base + j * 16)

    @pl.when(wid < B)
    def _():
        row_off = wid * RSTRIDE
        pltpu.sync_copy(mflat_hbm.at[pl.ds(wid * MBLK, MBLK)], mrow_ref)

        def bid_round(k, _):
            def pa(i, mm):
                return jnp.maximum(mm, jnp.max(mrow_ref[pl.ds(i * 16, 16)]))
            m = lax.fori_loop(0, MV, pa, NEG)

            def pb(i, jm):
                rv = mrow_ref[pl.ds(i * 16, 16)]
                cand = jnp.where(rv == m, i * 16 + iota, IMAX)
                return jnp.minimum(jm, jnp.min(cand))
            jmin = lax.fori_loop(0, MV, pb, IMAX)
            plsc.store_scatter(mrow_ref, [jnp.full((16,), jmin, jnp.int32)],
                               jnp.full((16,), NEG, jnp.float32), mask=lane0)
            bid_ref[k] = jmin
            return _
        lax.fori_loop(0, K, bid_round, 0)

        cps = []
        for k in range(K):
            cps.append(pltpu.make_async_copy(
                flat_hbm.at[pl.ds(row_off + bid_ref[k] * BLK, BLK)],
                gath_ref.at[pl.ds(k * BLK, BLK)], sem))
        waves = [cps[i:i + 8] for i in range(0, len(cps), 8)]
        for cp in waves[0]:
            cp.start()
        for w in range(1, len(waves)):
            for cp in waves[w]:
                cp.start()
            for cp in waves[w - 1]:
                cp.wait()
        for cp in waves[-1]:
            cp.wait()

        def init(i, c):
            pool_val[pl.ds(i * 16, 16)] = jnp.full((16,), NEG, jnp.float32)
            pool_idx[pl.ds(i * 16, 16)] = jnp.zeros((16,), jnp.int32)
            return c
        lax.fori_loop(0, PV, init, 0)
        cnt_ref[0] = jnp.int32(0)
        t_ref[0] = NEG

        def blk_body(k, carry):
            base = bid_ref[k] * BLK

            def grp(g, gc):
                scan_group(k * BLK + g * 128, base + g * 128)
                return gc
            lax.fori_loop(0, 8, grp, 0)
            return carry
        lax.fori_loop(0, K, blk_body, 0)

        extract32()
        v0 = wv_ref[pl.ds(0, 16)]
        v1 = wv_ref[pl.ds(16, 16)]
        mtop = v0[0]
        e0 = jnp.exp(v0 - mtop)
        e1 = jnp.exp(v1 - mtop)
        s = jnp.sum(e0) + jnp.sum(e1)
        prob_buf[pl.ds(0, 16)] = e0 / s
        prob_buf[pl.ds(16, 16)] = e1 / s
        pltpu.sync_copy(wi_ref, out_idx_hbm.at[pl.ds(wid * K, K)])
        pltpu.sync_copy(prob_buf, out_prob_hbm.at[pl.ds(wid * K, K)])


_mesh = plsc.VectorSubcoreMesh(core_axis_name="c", subcore_axis_name="s")

_select_call = functools.partial(
    pl.kernel,
    mesh=_mesh,
    compiler_params=pltpu.CompilerParams(needs_layout_passes=False),
    out_type=[
        jax.ShapeDtypeStruct((B * K,), jnp.int32),
        jax.ShapeDtypeStruct((B * K,), jnp.float32),
    ],
    scratch_types=[
        pltpu.VMEM((MBLK,), jnp.float32),
        pltpu.VMEM((K * BLK,), jnp.float32),
        pltpu.VMEM((POOL,), jnp.float32),
        pltpu.VMEM((POOL,), jnp.int32),
        pltpu.VMEM((K,), jnp.float32),
        pltpu.VMEM((K,), jnp.int32),
        pltpu.VMEM((K,), jnp.float32),
        pltpu.SMEM((1,), jnp.float32),
        pltpu.SMEM((1,), jnp.int32),
        pltpu.SMEM((K,), jnp.int32),
        pltpu.SemaphoreType.DMA,
    ],
)(_select_body)


def kernel(logits):
    flat4, m = _prep_call(logits)
    flat = flat4.reshape(-1)
    mrow = m.transpose(1, 0, 2).reshape(B, NBLK)
    idx_flat, prob_flat = _select_call(flat, mrow.reshape(-1))
    return idx_flat.reshape(B, K), prob_flat.reshape(B, K)


# speedup vs baseline: 2.0321x; 1.1409x over previous
import functools

import jax
import jax.numpy as jnp
import numpy as np
from jax import lax
from jax.experimental import pallas as pl
from jax.experimental.pallas import tpu as pltpu
from jax.experimental.pallas import tpu_sc as plsc

B = 32
N = 1_000_000
K = 32

BLK = 1024
SPAN = 65536
NSPAN = 16
RSTRIDE = NSPAN * SPAN
NBLK = NSPAN * 64
MBLK = 1024
TAIL_W = N - 15 * SPAN

POOL = 256
LIMIT = POOL - 16
PV = POOL // 16
MV = MBLK // 16

NEG = np.float32(-np.inf)
IMAX = np.int32(2**31 - 1)


def _prep_body(x_ref, f_ref, m_ref):
    c = pl.program_id(1)
    x = x_ref[...]
    col = lax.broadcasted_iota(jnp.int32, (8, SPAN), 1)
    x = jnp.where((c < NSPAN - 1) | (col < TAIL_W), x, NEG)
    f_ref[...] = x.reshape(8, 1, 512, 128)
    m_ref[...] = jnp.max(x.reshape(8, 64, BLK), axis=-1)[None]


_prep_call = pl.pallas_call(
    _prep_body,
    grid=(B // 8, NSPAN),
    in_specs=[pl.BlockSpec((8, SPAN), lambda rg, c: (rg, c))],
    out_specs=[
        pl.BlockSpec((8, 1, 512, 128), lambda rg, c: (rg, c, 0, 0)),
        pl.BlockSpec((1, 8, 64), lambda rg, c: (c, rg, 0)),
    ],
    out_shape=[
        jax.ShapeDtypeStruct((B, NSPAN, 512, 128), jnp.float32),
        jax.ShapeDtypeStruct((NSPAN, B, 64), jnp.float32),
    ],
)


def _select_body(flat_hbm, mflat_hbm, out_idx_hbm, out_prob_hbm,
                 mrow_ref, gath_ref, pool_val, pool_idx, wv_ref, wi_ref,
                 prob_buf, t_ref, cnt_ref, bid_ref, sem):
    nc = 2
    wid = lax.axis_index("s") * nc + lax.axis_index("c")
    iota = lax.iota(jnp.int32, 16)
    lane0 = iota == 0

    def extract32():
        def round_body(k, _):
            def pa(i, mm):
                return jnp.maximum(mm, jnp.max(pool_val[pl.ds(i * 16, 16)]))
            m = lax.fori_loop(0, PV, pa, NEG)

            def pb(i, jm):
                pv = pool_val[pl.ds(i * 16, 16)]
                pi = pool_idx[pl.ds(i * 16, 16)]
                cand = jnp.where(pv == m, pi, IMAX)
                return jnp.minimum(jm, jnp.min(cand))
            jmin = lax.fori_loop(0, PV, pb, IMAX)

            def pc(i, c):
                pv = pool_val[pl.ds(i * 16, 16)]
                pi = pool_idx[pl.ds(i * 16, 16)]
                pool_val[pl.ds(i * 16, 16)] = jnp.where(pi == jmin, NEG, pv)
                return c
            lax.fori_loop(0, PV, pc, 0)
            kv = jnp.full((16,), k, jnp.int32)
            plsc.store_scatter(wv_ref, [kv], jnp.full((16,), m, jnp.float32),
                               mask=lane0)
            plsc.store_scatter(wi_ref, [kv], jnp.full((16,), jmin, jnp.int32),
                               mask=lane0)
            return _
        lax.fori_loop(0, K, round_body, 0)

    def compact():
        extract32()
        for h in range(2):
            pool_val[pl.ds(h * 16, 16)] = wv_ref[pl.ds(h * 16, 16)]
            pool_idx[pl.ds(h * 16, 16)] = wi_ref[pl.ds(h * 16, 16)]

        def clear(i, c):
            pool_val[pl.ds(32 + i * 16, 16)] = jnp.full((16,), NEG, jnp.float32)
            return c
        lax.fori_loop(0, PV - 2, clear, 0)
        cnt_ref[0] = jnp.int32(K)
        t_ref[0] = wv_ref[pl.ds(K - 16, 16)][15]

    def process_vec(off, idx_base):
        v = gath_ref[pl.ds(off, 16)]
        m = v > t_ref[0]
        c = jnp.sum(m.astype(jnp.int32))

        @pl.when(c > 0)
        def _():
            cnt = cnt_ref[0]
            pos = cnt - 1 + plsc.cumsum(m.astype(jnp.int32))
            plsc.store_scatter(pool_val, [pos], v, mask=m)
            iv = idx_base + iota
            plsc.store_scatter(pool_idx, [pos], iv, mask=m)
            cnt_ref[0] = cnt + c

            @pl.when(cnt + c >= LIMIT)
            def _():
                compact()

    def scan_group(off, idx_base):
        gm = gath_ref[pl.ds(off, 16)]
        for j in range(1, 8):
            gm = jnp.maximum(gm, gath_ref[pl.ds(off + j * 16, 16)])

        @pl.when(jnp.max(gm) > t_ref[0])
        def _():
            for j in range(8):
                process_vec(off + j * 16, idx_base + j * 16)

    @pl.when(wid < B)
    def _():
        row_off = wid * RSTRIDE
        pltpu.sync_copy(mflat_hbm.at[pl.ds(wid * MBLK, MBLK)], mrow_ref)

        def bid_round(k, _):
            def pa(i, mm):
                return jnp.maximum(mm, jnp.max(mrow_ref[pl.ds(i * 16, 16)]))
            m = lax.fori_loop(0, MV, pa, NEG)

            def pb(i, jm):
                rv = mrow_ref[pl.ds(i * 16, 16)]
                cand = jnp.where(rv == m, i * 16 + iota, IMAX)
                return jnp.minimum(jm, jnp.min(cand))
            jmin = lax.fori_loop(0, MV, pb, IMAX)
            plsc.store_scatter(mrow_ref, [jnp.full((16,), jmin, jnp.int32)],
                               jnp.full((16,), NEG, jnp.float32), mask=lane0)
            bid_ref[k] = jmin
            return _
        lax.fori_loop(0, K, bid_round, 0)

        cps = []
        for k in range(K):
            cps.append(pltpu.make_async_copy(
                flat_hbm.at[pl.ds(row_off + bid_ref[k] * BLK, BLK)],
                gath_ref.at[pl.ds(k * BLK, BLK)], sem))
        waves = [cps[i:i + 8] for i in range(0, len(cps), 8)]
        for cp in waves[0]:
            cp.start()
        for w in range(1, len(waves)):
            for cp in waves[w]:
                cp.start()
            for cp in waves[w - 1]:
                cp.wait()
        for cp in waves[-1]:
            cp.wait()

        def init(i, c):
            pool_val[pl.ds(i * 16, 16)] = jnp.full((16,), NEG, jnp.float32)
            pool_idx[pl.ds(i * 16, 16)] = jnp.zeros((16,), jnp.int32)
            return c
        lax.fori_loop(0, PV, init, 0)
        cnt_ref[0] = jnp.int32(0)
        t_ref[0] = NEG

        def blk_body(k, carry):
            base = bid_ref[k] * BLK

            def grp(g, gc):
                scan_group(k * BLK + g * 128, base + g * 128)
                return gc
            lax.fori_loop(0, 8, grp, 0)
            return carry
        lax.fori_loop(0, K, blk_body, 0)

        extract32()
        v0 = wv_ref[pl.ds(0, 16)]
        v1 = wv_ref[pl.ds(16, 16)]
        mtop = v0[0]
        e0 = jnp.exp(v0 - mtop)
        e1 = jnp.exp(v1 - mtop)
        s = jnp.sum(e0) + jnp.sum(e1)
        prob_buf[pl.ds(0, 16)] = e0 / s
        prob_buf[pl.ds(16, 16)] = e1 / s
        pltpu.sync_copy(wi_ref, out_idx_hbm.at[pl.ds(wid * K, K)])
        pltpu.sync_copy(prob_buf, out_prob_hbm.at[pl.ds(wid * K, K)])


_mesh = plsc.VectorSubcoreMesh(core_axis_name="c", subcore_axis_name="s")

_select_call = functools.partial(
    pl.kernel,
    mesh=_mesh,
    compiler_params=pltpu.CompilerParams(needs_layout_passes=False),
    out_type=[
        jax.ShapeDtypeStruct((B * K,), jnp.int32),
        jax.ShapeDtypeStruct((B * K,), jnp.float32),
    ],
    scratch_types=[
        pltpu.VMEM((MBLK,), jnp.float32),
        pltpu.VMEM((K * BLK,), jnp.float32),
        pltpu.VMEM((POOL,), jnp.float32),
        pltpu.VMEM((POOL,), jnp.int32),
        pltpu.VMEM((K,), jnp.float32),
        pltpu.VMEM((K,), jnp.int32),
        pltpu.VMEM((K,), jnp.float32),
        pltpu.SMEM((1,), jnp.float32),
        pltpu.SMEM((1,), jnp.int32),
        pltpu.SMEM((K,), jnp.int32),
        pltpu.SemaphoreType.DMA,
    ],
)(_select_body)


def kernel(logits):
    flat4, m = _prep_call(logits)
    flat = flat4.reshape(-1)
    mrow = m.transpose(1, 0, 2).reshape(B, NBLK)
    idx_flat, prob_flat = _select_call(flat, mrow.reshape(-1))
    return idx_flat.reshape(B, K), prob_flat.reshape(B, K)


# speedup vs baseline: 2.2382x; 1.1014x over previous
import functools

import jax
import jax.numpy as jnp
import numpy as np
from jax import lax
from jax.experimental import pallas as pl
from jax.experimental.pallas import tpu as pltpu
from jax.experimental.pallas import tpu_sc as plsc

B = 32
N = 1_000_000
K = 32

BLK = 1024
SPAN = 131072
NSPAN = 8
RSTRIDE = NSPAN * SPAN
NBLK = NSPAN * 128
MBLK = 1024
TAIL_W = N - 7 * SPAN

POOL = 256
LIMIT = POOL - 16
PV = POOL // 16
MV = MBLK // 16

NEG = np.float32(-np.inf)
IMAX = np.int32(2**31 - 1)


def _prep_body(x_ref, f_ref, m_ref):
    c = pl.program_id(1)
    x = x_ref[...]
    col = lax.broadcasted_iota(jnp.int32, (8, SPAN), 1)
    x = jnp.where((c < NSPAN - 1) | (col < TAIL_W), x, NEG)
    f_ref[...] = x.reshape(8, 1, 1024, 128)
    m_ref[...] = jnp.max(x.reshape(8, 128, BLK), axis=-1)[None]


_prep_call = pl.pallas_call(
    _prep_body,
    grid=(B // 8, NSPAN),
    in_specs=[pl.BlockSpec((8, SPAN), lambda rg, c: (rg, c))],
    out_specs=[
        pl.BlockSpec((8, 1, 1024, 128), lambda rg, c: (rg, c, 0, 0)),
        pl.BlockSpec((1, 8, 128), lambda rg, c: (c, rg, 0)),
    ],
    out_shape=[
        jax.ShapeDtypeStruct((B, NSPAN, 1024, 128), jnp.float32),
        jax.ShapeDtypeStruct((NSPAN, B, 128), jnp.float32),
    ],
)


def _select_body(flat_hbm, mflat_hbm, out_idx_hbm, out_prob_hbm,
                 mrow_ref, gath_ref, pool_val, pool_idx, wv_ref, wi_ref,
                 prob_buf, t_ref, cnt_ref, bid_ref, sem):
    nc = 2
    wid = lax.axis_index("s") * nc + lax.axis_index("c")
    iota = lax.iota(jnp.int32, 16)
    lane0 = iota == 0

    def extract32():
        def round_body(k, _):
            def pa(i, mm):
                return jnp.maximum(mm, jnp.max(pool_val[pl.ds(i * 16, 16)]))
            m = lax.fori_loop(0, PV, pa, NEG)

            def pb(i, jm):
                pv = pool_val[pl.ds(i * 16, 16)]
                pi = pool_idx[pl.ds(i * 16, 16)]
                cand = jnp.where(pv == m, pi, IMAX)
                return jnp.minimum(jm, jnp.min(cand))
            jmin = lax.fori_loop(0, PV, pb, IMAX)

            def pc(i, c):
                pv = pool_val[pl.ds(i * 16, 16)]
                pi = pool_idx[pl.ds(i * 16, 16)]
                pool_val[pl.ds(i * 16, 16)] = jnp.where(pi == jmin, NEG, pv)
                return c
            lax.fori_loop(0, PV, pc, 0)
            kv = jnp.full((16,), k, jnp.int32)
            plsc.store_scatter(wv_ref, [kv], jnp.full((16,), m, jnp.float32),
                               mask=lane0)
            plsc.store_scatter(wi_ref, [kv], jnp.full((16,), jmin, jnp.int32),
                               mask=lane0)
            return _
        lax.fori_loop(0, K, round_body, 0)

    def compact():
        extract32()
        for h in range(2):
            pool_val[pl.ds(h * 16, 16)] = wv_ref[pl.ds(h * 16, 16)]
            pool_idx[pl.ds(h * 16, 16)] = wi_ref[pl.ds(h * 16, 16)]

        def clear(i, c):
            pool_val[pl.ds(32 + i * 16, 16)] = jnp.full((16,), NEG, jnp.float32)
            return c
        lax.fori_loop(0, PV - 2, clear, 0)
        cnt_ref[0] = jnp.int32(K)
        t_ref[0] = wv_ref[pl.ds(K - 16, 16)][15]

    def process_vec(off, idx_base):
        v = gath_ref[pl.ds(off, 16)]
        m = v > t_ref[0]
        c = jnp.sum(m.astype(jnp.int32))

        @pl.when(c > 0)
        def _():
            cnt = cnt_ref[0]
            pos = cnt - 1 + plsc.cumsum(m.astype(jnp.int32))
            plsc.store_scatter(pool_val, [pos], v, mask=m)
            iv = idx_base + iota
            plsc.store_scatter(pool_idx, [pos], iv, mask=m)
            cnt_ref[0] = cnt + c

            @pl.when(cnt + c >= LIMIT)
            def _():
                compact()

    def scan_group(off, idx_base):
        gm = gath_ref[pl.ds(off, 16)]
        for j in range(1, 8):
            gm = jnp.maximum(gm, gath_ref[pl.ds(off + j * 16, 16)])

        @pl.when(jnp.max(gm) > t_ref[0])
        def _():
            for j in range(8):
                process_vec(off + j * 16, idx_base + j * 16)

    @pl.when(wid < B)
    def _():
        row_off = wid * RSTRIDE
        pltpu.sync_copy(mflat_hbm.at[pl.ds(wid * MBLK, MBLK)], mrow_ref)

        def bid_round(k, _):
            def pa(i, mm):
                return jnp.maximum(mm, jnp.max(mrow_ref[pl.ds(i * 16, 16)]))
            m = lax.fori_loop(0, MV, pa, NEG)

            def pb(i, jm):
                rv = mrow_ref[pl.ds(i * 16, 16)]
                cand = jnp.where(rv == m, i * 16 + iota, IMAX)
                return jnp.minimum(jm, jnp.min(cand))
            jmin = lax.fori_loop(0, MV, pb, IMAX)
            plsc.store_scatter(mrow_ref, [jnp.full((16,), jmin, jnp.int32)],
                               jnp.full((16,), NEG, jnp.float32), mask=lane0)
            bid_ref[k] = jmin
            return _
        lax.fori_loop(0, K, bid_round, 0)

        cps = []
        for k in range(K):
            cps.append(pltpu.make_async_copy(
                flat_hbm.at[pl.ds(row_off + bid_ref[k] * BLK, BLK)],
                gath_ref.at[pl.ds(k * BLK, BLK)], sem))
        waves = [cps[i:i + 8] for i in range(0, len(cps), 8)]
        for cp in waves[0]:
            cp.start()
        for w in range(1, len(waves)):
            for cp in waves[w]:
                cp.start()
            for cp in waves[w - 1]:
                cp.wait()
        for cp in waves[-1]:
            cp.wait()

        def init(i, c):
            pool_val[pl.ds(i * 16, 16)] = jnp.full((16,), NEG, jnp.float32)
            pool_idx[pl.ds(i * 16, 16)] = jnp.zeros((16,), jnp.int32)
            return c
        lax.fori_loop(0, PV, init, 0)
        cnt_ref[0] = jnp.int32(0)
        t_ref[0] = NEG

        def blk_body(k, carry):
            base = bid_ref[k] * BLK

            def grp(g, gc):
                scan_group(k * BLK + g * 128, base + g * 128)
                return gc
            lax.fori_loop(0, 8, grp, 0)
            return carry
        lax.fori_loop(0, K, blk_body, 0)

        extract32()
        v0 = wv_ref[pl.ds(0, 16)]
        v1 = wv_ref[pl.ds(16, 16)]
        mtop = v0[0]
        e0 = jnp.exp(v0 - mtop)
        e1 = jnp.exp(v1 - mtop)
        s = jnp.sum(e0) + jnp.sum(e1)
        prob_buf[pl.ds(0, 16)] = e0 / s
        prob_buf[pl.ds(16, 16)] = e1 / s
        pltpu.sync_copy(wi_ref, out_idx_hbm.at[pl.ds(wid * K, K)])
        pltpu.sync_copy(prob_buf, out_prob_hbm.at[pl.ds(wid * K, K)])


_mesh = plsc.VectorSubcoreMesh(core_axis_name="c", subcore_axis_name="s")

_select_call = functools.partial(
    pl.kernel,
    mesh=_mesh,
    compiler_params=pltpu.CompilerParams(needs_layout_passes=False),
    out_type=[
        jax.ShapeDtypeStruct((B * K,), jnp.int32),
        jax.ShapeDtypeStruct((B * K,), jnp.float32),
    ],
    scratch_types=[
        pltpu.VMEM((MBLK,), jnp.float32),
        pltpu.VMEM((K * BLK,), jnp.float32),
        pltpu.VMEM((POOL,), jnp.float32),
        pltpu.VMEM((POOL,), jnp.int32),
        pltpu.VMEM((K,), jnp.float32),
        pltpu.VMEM((K,), jnp.int32),
        pltpu.VMEM((K,), jnp.float32),
        pltpu.SMEM((1,), jnp.float32),
        pltpu.SMEM((1,), jnp.int32),
        pltpu.SMEM((K,), jnp.int32),
        pltpu.SemaphoreType.DMA,
    ],
)(_select_body)


def kernel(logits):
    flat4, m = _prep_call(logits)
    flat = flat4.reshape(-1)
    mrow = m.transpose(1, 0, 2).reshape(B, NBLK)
    idx_flat, prob_flat = _select_call(flat, mrow.reshape(-1))
    return idx_flat.reshape(B, K), prob_flat.reshape(B, K)


# speedup vs baseline: 2.3258x; 1.0391x over previous
import functools

import jax
import jax.numpy as jnp
import numpy as np
from jax import lax
from jax.experimental import pallas as pl
from jax.experimental.pallas import tpu as pltpu
from jax.experimental.pallas import tpu_sc as plsc

B = 32
N = 1_000_000
K = 32

BLK = 1024
SPAN = 262144
NSPAN = 4
RSTRIDE = NSPAN * SPAN
NBLK = NSPAN * 256
MBLK = 1024
TAIL_W = N - 3 * SPAN

POOL = 256
LIMIT = POOL - 16
PV = POOL // 16
MV = MBLK // 16

NEG = np.float32(-np.inf)
IMAX = np.int32(2**31 - 1)


def _prep_body(x_ref, f_ref, m_ref):
    c = pl.program_id(1)
    x = x_ref[...]
    col = lax.broadcasted_iota(jnp.int32, (8, SPAN), 1)
    x = jnp.where((c < NSPAN - 1) | (col < TAIL_W), x, NEG)
    f_ref[...] = x.reshape(8, 1, 2048, 128)
    m_ref[...] = jnp.max(x.reshape(8, 256, BLK), axis=-1)[None]


_prep_call = pl.pallas_call(
    _prep_body,
    grid=(B // 8, NSPAN),
    in_specs=[pl.BlockSpec((8, SPAN), lambda rg, c: (rg, c))],
    out_specs=[
        pl.BlockSpec((8, 1, 2048, 128), lambda rg, c: (rg, c, 0, 0)),
        pl.BlockSpec((1, 8, 256), lambda rg, c: (c, rg, 0)),
    ],
    out_shape=[
        jax.ShapeDtypeStruct((B, NSPAN, 2048, 128), jnp.float32),
        jax.ShapeDtypeStruct((NSPAN, B, 256), jnp.float32),
    ],
)


def _select_body(flat_hbm, mflat_hbm, out_idx_hbm, out_prob_hbm,
                 mrow_ref, gath_ref, pool_val, pool_idx, wv_ref, wi_ref,
                 prob_buf, t_ref, cnt_ref, bid_ref, sem):
    nc = 2
    wid = lax.axis_index("s") * nc + lax.axis_index("c")
    iota = lax.iota(jnp.int32, 16)
    lane0 = iota == 0

    def extract32():
        def round_body(k, _):
            def pa(i, mm):
                return jnp.maximum(mm, jnp.max(pool_val[pl.ds(i * 16, 16)]))
            m = lax.fori_loop(0, PV, pa, NEG)

            def pb(i, jm):
                pv = pool_val[pl.ds(i * 16, 16)]
                pi = pool_idx[pl.ds(i * 16, 16)]
                cand = jnp.where(pv == m, pi, IMAX)
                return jnp.minimum(jm, jnp.min(cand))
            jmin = lax.fori_loop(0, PV, pb, IMAX)

            def pc(i, c):
                pv = pool_val[pl.ds(i * 16, 16)]
                pi = pool_idx[pl.ds(i * 16, 16)]
                pool_val[pl.ds(i * 16, 16)] = jnp.where(pi == jmin, NEG, pv)
                return c
            lax.fori_loop(0, PV, pc, 0)
            kv = jnp.full((16,), k, jnp.int32)
            plsc.store_scatter(wv_ref, [kv], jnp.full((16,), m, jnp.float32),
                               mask=lane0)
            plsc.store_scatter(wi_ref, [kv], jnp.full((16,), jmin, jnp.int32),
                               mask=lane0)
            return _
        lax.fori_loop(0, K, round_body, 0)

    def compact():
        extract32()
        for h in range(2):
            pool_val[pl.ds(h * 16, 16)] = wv_ref[pl.ds(h * 16, 16)]
            pool_idx[pl.ds(h * 16, 16)] = wi_ref[pl.ds(h * 16, 16)]

        def clear(i, c):
            pool_val[pl.ds(32 + i * 16, 16)] = jnp.full((16,), NEG, jnp.float32)
            return c
        lax.fori_loop(0, PV - 2, clear, 0)
        cnt_ref[0] = jnp.int32(K)
        t_ref[0] = wv_ref[pl.ds(K - 16, 16)][15]

    def process_vec(off, idx_base):
        v = gath_ref[pl.ds(off, 16)]
        m = v > t_ref[0]
        c = jnp.sum(m.astype(jnp.int32))

        @pl.when(c > 0)
        def _():
            cnt = cnt_ref[0]
            pos = cnt - 1 + plsc.cumsum(m.astype(jnp.int32))
            plsc.store_scatter(pool_val, [pos], v, mask=m)
            iv = idx_base + iota
            plsc.store_scatter(pool_idx, [pos], iv, mask=m)
            cnt_ref[0] = cnt + c

            @pl.when(cnt + c >= LIMIT)
            def _():
                compact()

    def scan_group(off, idx_base):
        gm = gath_ref[pl.ds(off, 16)]
        for j in range(1, 8):
            gm = jnp.maximum(gm, gath_ref[pl.ds(off + j * 16, 16)])

        @pl.when(jnp.max(gm) > t_ref[0])
        def _():
            for j in range(8):
                process_vec(off + j * 16, idx_base + j * 16)

    @pl.when(wid < B)
    def _():
        row_off = wid * RSTRIDE
        pltpu.sync_copy(mflat_hbm.at[pl.ds(wid * MBLK, MBLK)], mrow_ref)

        def bid_round(k, _):
            def pa(i, mm):
                return jnp.maximum(mm, jnp.max(mrow_ref[pl.ds(i * 16, 16)]))
            m = lax.fori_loop(0, MV, pa, NEG)

            def pb(i, jm):
                rv = mrow_ref[pl.ds(i * 16, 16)]
                cand = jnp.where(rv == m, i * 16 + iota, IMAX)
                return jnp.minimum(jm, jnp.min(cand))
            jmin = lax.fori_loop(0, MV, pb, IMAX)
            plsc.store_scatter(mrow_ref, [jnp.full((16,), jmin, jnp.int32)],
                               jnp.full((16,), NEG, jnp.float32), mask=lane0)
            bid_ref[k] = jmin
            return _
        lax.fori_loop(0, K, bid_round, 0)

        cps = []
        for k in range(K):
            cps.append(pltpu.make_async_copy(
                flat_hbm.at[pl.ds(row_off + bid_ref[k] * BLK, BLK)],
                gath_ref.at[pl.ds(k * BLK, BLK)], sem))
        waves = [cps[i:i + 8] for i in range(0, len(cps), 8)]
        for cp in waves[0]:
            cp.start()
        for w in range(1, len(waves)):
            for cp in waves[w]:
                cp.start()
            for cp in waves[w - 1]:
                cp.wait()
        for cp in waves[-1]:
            cp.wait()

        def init(i, c):
            pool_val[pl.ds(i * 16, 16)] = jnp.full((16,), NEG, jnp.float32)
            pool_idx[pl.ds(i * 16, 16)] = jnp.zeros((16,), jnp.int32)
            return c
        lax.fori_loop(0, PV, init, 0)
        cnt_ref[0] = jnp.int32(0)
        t_ref[0] = NEG

        def blk_body(k, carry):
            base = bid_ref[k] * BLK

            def grp(g, gc):
                scan_group(k * BLK + g * 128, base + g * 128)
                return gc
            lax.fori_loop(0, 8, grp, 0)
            return carry
        lax.fori_loop(0, K, blk_body, 0)

        extract32()
        v0 = wv_ref[pl.ds(0, 16)]
        v1 = wv_ref[pl.ds(16, 16)]
        mtop = v0[0]
        e0 = jnp.exp(v0 - mtop)
        e1 = jnp.exp(v1 - mtop)
        s = jnp.sum(e0) + jnp.sum(e1)
        prob_buf[pl.ds(0, 16)] = e0 / s
        prob_buf[pl.ds(16, 16)] = e1 / s
        pltpu.sync_copy(wi_ref, out_idx_hbm.at[pl.ds(wid * K, K)])
        pltpu.sync_copy(prob_buf, out_prob_hbm.at[pl.ds(wid * K, K)])


_mesh = plsc.VectorSubcoreMesh(core_axis_name="c", subcore_axis_name="s")

_select_call = functools.partial(
    pl.kernel,
    mesh=_mesh,
    compiler_params=pltpu.CompilerParams(needs_layout_passes=False),
    out_type=[
        jax.ShapeDtypeStruct((B * K,), jnp.int32),
        jax.ShapeDtypeStruct((B * K,), jnp.float32),
    ],
    scratch_types=[
        pltpu.VMEM((MBLK,), jnp.float32),
        pltpu.VMEM((K * BLK,), jnp.float32),
        pltpu.VMEM((POOL,), jnp.float32),
        pltpu.VMEM((POOL,), jnp.int32),
        pltpu.VMEM((K,), jnp.float32),
        pltpu.VMEM((K,), jnp.int32),
        pltpu.VMEM((K,), jnp.float32),
        pltpu.SMEM((1,), jnp.float32),
        pltpu.SMEM((1,), jnp.int32),
        pltpu.SMEM((K,), jnp.int32),
        pltpu.SemaphoreType.DMA,
    ],
)(_select_body)


def kernel(logits):
    flat4, m = _prep_call(logits)
    flat = flat4.reshape(-1)
    mrow = m.transpose(1, 0, 2).reshape(B, NBLK)
    idx_flat, prob_flat = _select_call(flat, mrow.reshape(-1))
    return idx_flat.reshape(B, K), prob_flat.reshape(B, K)


# speedup vs baseline: 2.4455x; 1.0515x over previous
import functools

import jax
import jax.numpy as jnp
import numpy as np
from jax import lax
from jax.experimental import pallas as pl
from jax.experimental.pallas import tpu as pltpu
from jax.experimental.pallas import tpu_sc as plsc

B = 32
N = 1_000_000
K = 32

BLK = 1024
SPAN = 262144
NSPAN = 4
RSTRIDE = NSPAN * SPAN
NBLK = NSPAN * 256
MBLK = 1024
TAIL_W = N - 3 * SPAN

POOL = 256
LIMIT = POOL - 16
PV = POOL // 16
MV = MBLK // 16

NEG = np.float32(-np.inf)
IMAX = np.int32(2**31 - 1)


def _prep_body(x_ref, f_ref, m_ref):
    c = pl.program_id(1)
    x = x_ref[...]
    col = lax.broadcasted_iota(jnp.int32, (8, SPAN), 1)
    x = jnp.where((c < NSPAN - 1) | (col < TAIL_W), x, NEG)
    f_ref[...] = x.reshape(8, 1, 2048, 128)
    m_ref[...] = jnp.max(x.reshape(8, 256, BLK), axis=-1)[None]


_prep_call = pl.pallas_call(
    _prep_body,
    grid=(B // 8, NSPAN),
    in_specs=[pl.BlockSpec((8, SPAN), lambda rg, c: (rg, c))],
    out_specs=[
        pl.BlockSpec((8, 1, 2048, 128), lambda rg, c: (rg, c, 0, 0)),
        pl.BlockSpec((1, 8, 256), lambda rg, c: (c, rg, 0)),
    ],
    out_shape=[
        jax.ShapeDtypeStruct((B, NSPAN, 2048, 128), jnp.float32),
        jax.ShapeDtypeStruct((NSPAN, B, 256), jnp.float32),
    ],
)


def _select_body(flat_hbm, mflat_hbm, out_idx_hbm, out_prob_hbm,
                 mrow_ref, gath_ref, pool_val, pool_idx, wv_ref, wi_ref,
                 prob_buf, t_ref, cnt_ref, bid_ref, sem):
    nc = 2
    wid = lax.axis_index("s") * nc + lax.axis_index("c")
    iota = lax.iota(jnp.int32, 16)
    lane0 = iota == 0

    def extract32():
        def round_body(k, _):
            def pa(i, mm):
                return jnp.maximum(mm, jnp.max(pool_val[pl.ds(i * 16, 16)]))
            m = lax.fori_loop(0, PV, pa, NEG)

            def pb(i, jm):
                pv = pool_val[pl.ds(i * 16, 16)]
                pi = pool_idx[pl.ds(i * 16, 16)]
                cand = jnp.where(pv == m, pi, IMAX)
                return jnp.minimum(jm, jnp.min(cand))
            jmin = lax.fori_loop(0, PV, pb, IMAX)

            def pc(i, c):
                pv = pool_val[pl.ds(i * 16, 16)]
                pi = pool_idx[pl.ds(i * 16, 16)]
                pool_val[pl.ds(i * 16, 16)] = jnp.where(pi == jmin, NEG, pv)
                return c
            lax.fori_loop(0, PV, pc, 0)
            kv = jnp.full((16,), k, jnp.int32)
            plsc.store_scatter(wv_ref, [kv], jnp.full((16,), m, jnp.float32),
                               mask=lane0)
            plsc.store_scatter(wi_ref, [kv], jnp.full((16,), jmin, jnp.int32),
                               mask=lane0)
            return _
        lax.fori_loop(0, K, round_body, 0)

    def compact():
        extract32()
        for h in range(2):
            pool_val[pl.ds(h * 16, 16)] = wv_ref[pl.ds(h * 16, 16)]
            pool_idx[pl.ds(h * 16, 16)] = wi_ref[pl.ds(h * 16, 16)]

        def clear(i, c):
            pool_val[pl.ds(32 + i * 16, 16)] = jnp.full((16,), NEG, jnp.float32)
            return c
        lax.fori_loop(0, PV - 2, clear, 0)
        cnt_ref[0] = jnp.int32(K)
        t_ref[0] = wv_ref[pl.ds(K - 16, 16)][15]

    def process_vec(src_ref, off, idx_base):
        v = src_ref[pl.ds(off, 16)]
        m = v > t_ref[0]
        c = jnp.sum(m.astype(jnp.int32))

        @pl.when(c > 0)
        def _():
            cnt = cnt_ref[0]
            pos = cnt - 1 + plsc.cumsum(m.astype(jnp.int32))
            plsc.store_scatter(pool_val, [pos], v, mask=m)
            iv = idx_base + iota
            plsc.store_scatter(pool_idx, [pos], iv, mask=m)
            cnt_ref[0] = cnt + c

            @pl.when(cnt + c >= LIMIT)
            def _():
                compact()

    def scan_group(src_ref, off, idx_base):
        gm = src_ref[pl.ds(off, 16)]
        for j in range(1, 8):
            gm = jnp.maximum(gm, src_ref[pl.ds(off + j * 16, 16)])

        @pl.when(jnp.max(gm) > t_ref[0])
        def _():
            for j in range(8):
                process_vec(src_ref, off + j * 16, idx_base + j * 16)

    def reset_pool():
        def init(i, c):
            pool_val[pl.ds(i * 16, 16)] = jnp.full((16,), NEG, jnp.float32)
            pool_idx[pl.ds(i * 16, 16)] = jnp.zeros((16,), jnp.int32)
            return c
        lax.fori_loop(0, PV, init, 0)
        cnt_ref[0] = jnp.int32(0)
        t_ref[0] = NEG

    @pl.when(wid < B)
    def _():
        row_off = wid * RSTRIDE
        pltpu.sync_copy(mflat_hbm.at[pl.ds(wid * MBLK, MBLK)], mrow_ref)

        reset_pool()

        def mgrp(g, gc):
            scan_group(mrow_ref, g * 128, g * 128)
            return gc
        lax.fori_loop(0, MV // 8, mgrp, 0)
        extract32()
        w0 = wi_ref[pl.ds(0, 16)]
        w1 = wi_ref[pl.ds(16, 16)]
        for j in range(16):
            bid_ref[j] = w0[j]
            bid_ref[16 + j] = w1[j]

        cps = []
        for k in range(K):
            cps.append(pltpu.make_async_copy(
                flat_hbm.at[pl.ds(row_off + bid_ref[k] * BLK, BLK)],
                gath_ref.at[pl.ds(k * BLK, BLK)], sem))
        waves = [cps[i:i + 8] for i in range(0, len(cps), 8)]
        for cp in waves[0]:
            cp.start()
        for w in range(1, len(waves)):
            for cp in waves[w]:
                cp.start()
            for cp in waves[w - 1]:
                cp.wait()
        for cp in waves[-1]:
            cp.wait()

        reset_pool()

        def blk_body(k, carry):
            base = bid_ref[k] * BLK

            def grp(g, gc):
                scan_group(gath_ref, k * BLK + g * 128, base + g * 128)
                return gc
            lax.fori_loop(0, 8, grp, 0)
            return carry
        lax.fori_loop(0, K, blk_body, 0)

        extract32()
        v0 = wv_ref[pl.ds(0, 16)]
        v1 = wv_ref[pl.ds(16, 16)]
        mtop = v0[0]
        e0 = jnp.exp(v0 - mtop)
        e1 = jnp.exp(v1 - mtop)
        s = jnp.sum(e0) + jnp.sum(e1)
        prob_buf[pl.ds(0, 16)] = e0 / s
        prob_buf[pl.ds(16, 16)] = e1 / s
        pltpu.sync_copy(wi_ref, out_idx_hbm.at[pl.ds(wid * K, K)])
        pltpu.sync_copy(prob_buf, out_prob_hbm.at[pl.ds(wid * K, K)])


_mesh = plsc.VectorSubcoreMesh(core_axis_name="c", subcore_axis_name="s")

_select_call = functools.partial(
    pl.kernel,
    mesh=_mesh,
    compiler_params=pltpu.CompilerParams(needs_layout_passes=False),
    out_type=[
        jax.ShapeDtypeStruct((B * K,), jnp.int32),
        jax.ShapeDtypeStruct((B * K,), jnp.float32),
    ],
    scratch_types=[
        pltpu.VMEM((MBLK,), jnp.float32),
        pltpu.VMEM((K * BLK,), jnp.float32),
        pltpu.VMEM((POOL,), jnp.float32),
        pltpu.VMEM((POOL,), jnp.int32),
        pltpu.VMEM((K,), jnp.float32),
        pltpu.VMEM((K,), jnp.int32),
        pltpu.VMEM((K,), jnp.float32),
        pltpu.SMEM((1,), jnp.float32),
        pltpu.SMEM((1,), jnp.int32),
        pltpu.SMEM((K,), jnp.int32),
        pltpu.SemaphoreType.DMA,
    ],
)(_select_body)


def kernel(logits):
    flat4, m = _prep_call(logits)
    flat = flat4.reshape(-1)
    mrow = m.transpose(1, 0, 2).reshape(B, NBLK)
    idx_flat, prob_flat = _select_call(flat, mrow.reshape(-1))
    return idx_flat.reshape(B, K), prob_flat.reshape(B, K)


# speedup vs baseline: 2.5785x; 1.0544x over previous
import functools

import jax
import jax.numpy as jnp
import numpy as np
from jax import lax
from jax.experimental import pallas as pl
from jax.experimental.pallas import tpu as pltpu
from jax.experimental.pallas import tpu_sc as plsc

B = 32
N = 1_000_000
K = 32

BLK = 512
SPAN = 262144
NSPAN = 4
RSTRIDE = NSPAN * SPAN
NBLK = NSPAN * 512
MBLK = 2048
TAIL_W = N - 3 * SPAN

POOL = 256
LIMIT = POOL - 16
PV = POOL // 16
MV = MBLK // 16

NEG = np.float32(-np.inf)
IMAX = np.int32(2**31 - 1)


def _prep_body(x_ref, f_ref, m_ref):
    c = pl.program_id(1)
    x = x_ref[...]
    col = lax.broadcasted_iota(jnp.int32, (8, SPAN), 1)
    x = jnp.where((c < NSPAN - 1) | (col < TAIL_W), x, NEG)
    f_ref[...] = x.reshape(8, 1, 2048, 128)
    m_ref[...] = jnp.max(x.reshape(8, 512, BLK), axis=-1)[None]


_prep_call = pl.pallas_call(
    _prep_body,
    grid=(B // 8, NSPAN),
    in_specs=[pl.BlockSpec((8, SPAN), lambda rg, c: (rg, c))],
    out_specs=[
        pl.BlockSpec((8, 1, 2048, 128), lambda rg, c: (rg, c, 0, 0)),
        pl.BlockSpec((1, 8, 512), lambda rg, c: (c, rg, 0)),
    ],
    out_shape=[
        jax.ShapeDtypeStruct((B, NSPAN, 2048, 128), jnp.float32),
        jax.ShapeDtypeStruct((NSPAN, B, 512), jnp.float32),
    ],
)


def _select_body(flat_hbm, mflat_hbm, out_idx_hbm, out_prob_hbm,
                 mrow_ref, gath_ref, pool_val, pool_idx, wv_ref, wi_ref,
                 prob_buf, t_ref, cnt_ref, bid_ref, sem):
    nc = 2
    wid = lax.axis_index("s") * nc + lax.axis_index("c")
    iota = lax.iota(jnp.int32, 16)
    lane0 = iota == 0

    def extract32():
        def round_body(k, _):
            def pa(i, mm):
                return jnp.maximum(mm, jnp.max(pool_val[pl.ds(i * 16, 16)]))
            m = lax.fori_loop(0, PV, pa, NEG)

            def pb(i, jm):
                pv = pool_val[pl.ds(i * 16, 16)]
                pi = pool_idx[pl.ds(i * 16, 16)]
                cand = jnp.where(pv == m, pi, IMAX)
                return jnp.minimum(jm, jnp.min(cand))
            jmin = lax.fori_loop(0, PV, pb, IMAX)

            def pc(i, c):
                pv = pool_val[pl.ds(i * 16, 16)]
                pi = pool_idx[pl.ds(i * 16, 16)]
                pool_val[pl.ds(i * 16, 16)] = jnp.where(pi == jmin, NEG, pv)
                return c
            lax.fori_loop(0, PV, pc, 0)
            kv = jnp.full((16,), k, jnp.int32)
            plsc.store_scatter(wv_ref, [kv], jnp.full((16,), m, jnp.float32),
                               mask=lane0)
            plsc.store_scatter(wi_ref, [kv], jnp.full((16,), jmin, jnp.int32),
                               mask=lane0)
            return _
        lax.fori_loop(0, K, round_body, 0)

    def compact():
        extract32()
        for h in range(2):
            pool_val[pl.ds(h * 16, 16)] = wv_ref[pl.ds(h * 16, 16)]
            pool_idx[pl.ds(h * 16, 16)] = wi_ref[pl.ds(h * 16, 16)]

        def clear(i, c):
            pool_val[pl.ds(32 + i * 16, 16)] = jnp.full((16,), NEG, jnp.float32)
            return c
        lax.fori_loop(0, PV - 2, clear, 0)
        cnt_ref[0] = jnp.int32(K)
        t_ref[0] = wv_ref[pl.ds(K - 16, 16)][15]

    def process_vec(src_ref, off, idx_base):
        v = src_ref[pl.ds(off, 16)]
        m = v > t_ref[0]
        c = jnp.sum(m.astype(jnp.int32))

        @pl.when(c > 0)
        def _():
            cnt = cnt_ref[0]
            pos = cnt - 1 + plsc.cumsum(m.astype(jnp.int32))
            plsc.store_scatter(pool_val, [pos], v, mask=m)
            iv = idx_base + iota
            plsc.store_scatter(pool_idx, [pos], iv, mask=m)
            cnt_ref[0] = cnt + c

            @pl.when(cnt + c >= LIMIT)
            def _():
                compact()

    def scan_group(src_ref, off, idx_base):
        gm = src_ref[pl.ds(off, 16)]
        for j in range(1, 8):
            gm = jnp.maximum(gm, src_ref[pl.ds(off + j * 16, 16)])

        @pl.when(jnp.max(gm) > t_ref[0])
        def _():
            for j in range(8):
                process_vec(src_ref, off + j * 16, idx_base + j * 16)

    def reset_pool():
        def init(i, c):
            pool_val[pl.ds(i * 16, 16)] = jnp.full((16,), NEG, jnp.float32)
            pool_idx[pl.ds(i * 16, 16)] = jnp.zeros((16,), jnp.int32)
            return c
        lax.fori_loop(0, PV, init, 0)
        cnt_ref[0] = jnp.int32(0)
        t_ref[0] = NEG

    @pl.when(wid < B)
    def _():
        row_off = wid * RSTRIDE
        pltpu.sync_copy(mflat_hbm.at[pl.ds(wid * MBLK, MBLK)], mrow_ref)

        reset_pool()

        def mgrp(g, gc):
            scan_group(mrow_ref, g * 128, g * 128)
            return gc
        lax.fori_loop(0, MV // 8, mgrp, 0)
        extract32()
        w0 = wi_ref[pl.ds(0, 16)]
        w1 = wi_ref[pl.ds(16, 16)]
        for j in range(16):
            bid_ref[j] = w0[j]
            bid_ref[16 + j] = w1[j]

        cps = []
        for k in range(K):
            cps.append(pltpu.make_async_copy(
                flat_hbm.at[pl.ds(row_off + bid_ref[k] * BLK, BLK)],
                gath_ref.at[pl.ds(k * BLK, BLK)], sem))
        waves = [cps[i:i + 8] for i in range(0, len(cps), 8)]
        for cp in waves[0]:
            cp.start()
        for w in range(1, len(waves)):
            for cp in waves[w]:
                cp.start()
            for cp in waves[w - 1]:
                cp.wait()
        for cp in waves[-1]:
            cp.wait()

        reset_pool()

        def blk_body(k, carry):
            base = bid_ref[k] * BLK

            def grp(g, gc):
                scan_group(gath_ref, k * BLK + g * 128, base + g * 128)
                return gc
            lax.fori_loop(0, BLK // 128, grp, 0)
            return carry
        lax.fori_loop(0, K, blk_body, 0)

        extract32()
        v0 = wv_ref[pl.ds(0, 16)]
        v1 = wv_ref[pl.ds(16, 16)]
        mtop = v0[0]
        e0 = jnp.exp(v0 - mtop)
        e1 = jnp.exp(v1 - mtop)
        s = jnp.sum(e0) + jnp.sum(e1)
        prob_buf[pl.ds(0, 16)] = e0 / s
        prob_buf[pl.ds(16, 16)] = e1 / s
        pltpu.sync_copy(wi_ref, out_idx_hbm.at[pl.ds(wid * K, K)])
        pltpu.sync_copy(prob_buf, out_prob_hbm.at[pl.ds(wid * K, K)])


_mesh = plsc.VectorSubcoreMesh(core_axis_name="c", subcore_axis_name="s")

_select_call = functools.partial(
    pl.kernel,
    mesh=_mesh,
    compiler_params=pltpu.CompilerParams(needs_layout_passes=False),
    out_type=[
        jax.ShapeDtypeStruct((B * K,), jnp.int32),
        jax.ShapeDtypeStruct((B * K,), jnp.float32),
    ],
    scratch_types=[
        pltpu.VMEM((MBLK,), jnp.float32),
        pltpu.VMEM((K * BLK,), jnp.float32),
        pltpu.VMEM((POOL,), jnp.float32),
        pltpu.VMEM((POOL,), jnp.int32),
        pltpu.VMEM((K,), jnp.float32),
        pltpu.VMEM((K,), jnp.int32),
        pltpu.VMEM((K,), jnp.float32),
        pltpu.SMEM((1,), jnp.float32),
        pltpu.SMEM((1,), jnp.int32),
        pltpu.SMEM((K,), jnp.int32),
        pltpu.SemaphoreType.DMA,
    ],
)(_select_body)


def kernel(logits):
    flat4, m = _prep_call(logits)
    flat = flat4.reshape(-1)
    mrow = m.transpose(1, 0, 2).reshape(B, NBLK)
    idx_flat, prob_flat = _select_call(flat, mrow.reshape(-1))
    return idx_flat.reshape(B, K), prob_flat.reshape(B, K)
